# baseline (speedup 1.0000x reference)
import functools

import jax
import jax.numpy as jnp
from jax import lax
from jax.experimental import pallas as pl
from jax.experimental.pallas import tpu as pltpu
from jax.experimental.pallas import tpu_sc as plsc

NC = 2
NS = 16
NW = NC * NS
CHUNK = 64
NODE_PAD = 10240
RPT = NODE_PAD // NS
BN_SCALE = 1.0 / (1.0 + 1e-5) ** 0.5


CNT_W = 8


HROWS = NODE_PAD * CNT_W // 128


PACK_SHIFT = 14


def _sc_count(packed_flat, iota_rows, zeros_c):
    epw = packed_flat.shape[1]
    slc = HROWS // NS
    mrows = 128

    @functools.partial(
        pl.kernel,
        out_type=jax.ShapeDtypeStruct((NC, HROWS, 128), jnp.float32),
        mesh=plsc.VectorSubcoreMesh(core_axis_name="c", subcore_axis_name="s"),
        scratch_types=[
            pltpu.VMEM_SHARED((HROWS, 128), jnp.float32),
            pltpu.VMEM((HROWS, 128), jnp.float32),
            pltpu.VMEM((epw,), jnp.int32),
            pltpu.VMEM((HROWS // mrows, mrows), jnp.int32),
        ],
        compiler_params=pltpu.CompilerParams(needs_layout_passes=False),
    )
    def k(pk_hbm, iota_hbm, zeros_hbm, cnt_hbm, cnt_sh, hist, idx_v, rowi_v):
        c = lax.axis_index("c")
        s = lax.axis_index("s")
        w = c * NS + s
        r0 = s * slc
        pltpu.sync_copy(zeros_hbm.at[pl.ds(0, slc)], cnt_sh.at[pl.ds(r0, slc)])
        pltpu.sync_copy(zeros_hbm, hist)
        pltpu.sync_copy(pk_hbm.at[w], idx_v)
        pltpu.sync_copy(iota_hbm, rowi_v)

        ones_v = jnp.ones((16,), jnp.float32)

        def body(j, carry):
            pk = idx_v[pl.ds(j * 16, 16)]
            pos = lax.shift_right_logical(pk, PACK_SHIFT) * CNT_W
            row = lax.shift_right_logical(pos, 7)
            col = lax.bitwise_and(pos, 127)
            plsc.addupdate_scatter(hist, [row, col], ones_v)
            return carry

        lax.fori_loop(0, epw // 16, body, 0)
        plsc.subcore_barrier()

        def mbody(g, carry):
            pltpu.sync_copy(hist.at[pl.ds(g * mrows, mrows)],
                            cnt_sh.at[rowi_v.at[g]], add=True)
            return carry

        lax.fori_loop(0, HROWS // mrows, mbody, 0)
        plsc.subcore_barrier()
        pltpu.sync_copy(cnt_sh.at[pl.ds(r0, slc)],
                        cnt_hbm.at[c, pl.ds(r0, slc)])

    return k(packed_flat, iota_rows, zeros_c)


def _unpack_chunk(idx_v, j, sidx, didx):
    for l in range(CHUNK // 16):
        pk = idx_v[pl.ds(j * CHUNK + l * 16, 16)]
        sidx[pl.ds(l * 16, 16)] = lax.bitwise_and(pk, (1 << PACK_SHIFT) - 1)
        didx[pl.ds(l * 16, 16)] = lax.shift_right_logical(pk, PACK_SHIFT)


def _sc_agg(u, packed_s, zeros_c):
    epw = packed_s.shape[1]
    nchunk = epw // CHUNK
    d = u.shape[1]
    dh = d // 2
    assert nchunk % 2 == 0
    npair = nchunk // 2

    @functools.partial(
        pl.kernel,
        out_type=jax.ShapeDtypeStruct((NODE_PAD, d), jnp.float32),
        mesh=plsc.VectorSubcoreMesh(core_axis_name="c", subcore_axis_name="s"),
        compiler_params=pltpu.CompilerParams(use_tc_tiling_on_sc=False),
        scratch_types=[
            pltpu.VMEM_SHARED((NODE_PAD, dh), jnp.float32),
            pltpu.VMEM_SHARED((NODE_PAD, dh), jnp.float32),
            pltpu.VMEM((epw,), jnp.int32),
            pltpu.VMEM((CHUNK,), jnp.int32),
            pltpu.VMEM((CHUNK,), jnp.int32),
            pltpu.VMEM((CHUNK,), jnp.int32),
            pltpu.VMEM((CHUNK,), jnp.int32),
            pltpu.VMEM((CHUNK, dh), jnp.float32),
            pltpu.VMEM((CHUNK, dh), jnp.float32),
            pltpu.SemaphoreType.DMA,
            pltpu.SemaphoreType.DMA,
        ],
    )
    def k(u_hbm, pk_hbm, z_hbm, agg_hbm, u_sh, agg_sh,
          idx_v, sidx0, didx0, sidx1, didx1, rows0, rows1, sem0, sem1):
        c = lax.axis_index("c")
        s = lax.axis_index("s")
        r0 = s * RPT
        c0 = c * dh
        pltpu.sync_copy(u_hbm.at[pl.ds(r0, RPT), pl.ds(c0, dh)],
                        u_sh.at[pl.ds(r0, RPT)])
        pltpu.sync_copy(z_hbm, agg_sh.at[pl.ds(r0, RPT)])
        pltpu.sync_copy(pk_hbm.at[s], idx_v)
        plsc.subcore_barrier()

        _unpack_chunk(idx_v, 0, sidx0, didx0)
        pltpu.async_copy(u_sh.at[sidx0], rows0, sem0)

        def body(i, carry):
            j = 2 * i
            _unpack_chunk(idx_v, j + 1, sidx1, didx1)
            pltpu.async_copy(u_sh.at[sidx1], rows1, sem1)
            pltpu.make_async_copy(u_sh.at[sidx0], rows0, sem0).wait()
            pltpu.sync_copy(rows0, agg_sh.at[didx0], add=True)

            @pl.when(i + 1 < npair)
            def _():
                _unpack_chunk(idx_v, j + 2, sidx0, didx0)
                pltpu.async_copy(u_sh.at[sidx0], rows0, sem0)

            pltpu.make_async_copy(u_sh.at[sidx1], rows1, sem1).wait()
            pltpu.sync_copy(rows1, agg_sh.at[didx1], add=True)
            return carry

        lax.fori_loop(0, npair, body, 0)
        plsc.subcore_barrier()
        pltpu.sync_copy(agg_sh.at[pl.ds(r0, RPT)],
                        agg_hbm.at[pl.ds(r0, RPT), pl.ds(c0, dh)])

    return k(u, packed_s, zeros_c)


BLK = 256


def _dinv_of(cnt_ref):
    return lax.rsqrt(cnt_ref[0, :, :1] + cnt_ref[1, :, :1] + 1.0)


def _tc_mm(x_pad, W0):
    d_in = x_pad.shape[1]
    d_out = W0.shape[1]

    def body(x_ref, w_ref, o_ref):
        o_ref[...] = jnp.dot(x_ref[...], w_ref[...],
                             preferred_element_type=jnp.float32)

    return pl.pallas_call(
        body,
        grid=(NODE_PAD // BLK,),
        in_specs=[
            pl.BlockSpec((BLK, d_in), lambda r: (r, 0)),
            pl.BlockSpec((d_in, d_out), lambda r: (0, 0)),
        ],
        out_specs=pl.BlockSpec((BLK, d_out), lambda r: (r, 0)),
        out_shape=jax.ShapeDtypeStruct((NODE_PAD, d_out), jnp.float32),
    )(x_pad, W0)


def _tc_scale(t0, cnt):
    d_out = t0.shape[1]

    def body(t_ref, cnt_ref, o_ref):
        o_ref[...] = t_ref[...] * _dinv_of(cnt_ref)

    return pl.pallas_call(
        body,
        grid=(NODE_PAD // BLK,),
        in_specs=[
            pl.BlockSpec((BLK, d_out), lambda r: (r, 0)),
            pl.BlockSpec((NC, BLK, CNT_W), lambda r: (0, r, 0)),
        ],
        out_specs=pl.BlockSpec((BLK, d_out), lambda r: (r, 0)),
        out_shape=jax.ShapeDtypeStruct((NODE_PAD, d_out), jnp.float32),
    )(t0, cnt)


def _tc_layer(agg, u_prev, cnt, b, g, be, W):
    d = u_prev.shape[1]
    d_out = W.shape[1]

    def body(a_ref, u_ref, cnt_ref, b_ref, g_ref, be_ref, w_ref, o_ref):
        dinv = _dinv_of(cnt_ref)
        z = dinv * (a_ref[...] + u_ref[...]) + b_ref[...]
        z = z * (g_ref[...] * BN_SCALE) + be_ref[...]
        z = jnp.maximum(z, 0.0)
        h = jnp.dot(z, w_ref[...], preferred_element_type=jnp.float32)
        o_ref[...] = h * dinv

    return pl.pallas_call(
        body,
        grid=(NODE_PAD // BLK,),
        in_specs=[
            pl.BlockSpec((BLK, d), lambda r: (r, 0)),
            pl.BlockSpec((BLK, d), lambda r: (r, 0)),
            pl.BlockSpec((NC, BLK, CNT_W), lambda r: (0, r, 0)),
            pl.BlockSpec((d,), lambda r: (0,)),
            pl.BlockSpec((d,), lambda r: (0,)),
            pl.BlockSpec((d,), lambda r: (0,)),
            pl.BlockSpec((d, d_out), lambda r: (0, 0)),
        ],
        out_specs=pl.BlockSpec((BLK, d_out), lambda r: (r, 0)),
        out_shape=jax.ShapeDtypeStruct((NODE_PAD, d_out), jnp.float32),
    )(agg, u_prev, cnt, b, g, be, W)


def _tc_out(agg, u_prev, cnt, b_pad, n_rows):
    d = u_prev.shape[1]
    blk = 400

    def body(a_ref, u_ref, cnt_ref, b_ref, o_ref):
        dinv = _dinv_of(cnt_ref)
        z = dinv * (a_ref[...] + u_ref[...]) + b_ref[...]
        m = jnp.max(z, axis=1, keepdims=True)
        t = z - m
        lse = jnp.log(jnp.sum(jnp.exp(t), axis=1, keepdims=True))
        o_ref[...] = t - lse

    return pl.pallas_call(
        body,
        grid=(n_rows // blk,),
        in_specs=[
            pl.BlockSpec((blk, d), lambda r: (r, 0)),
            pl.BlockSpec((blk, d), lambda r: (r, 0)),
            pl.BlockSpec((NC, blk, CNT_W), lambda r: (0, r, 0)),
            pl.BlockSpec((d,), lambda r: (0,)),
        ],
        out_specs=pl.BlockSpec((blk, d), lambda r: (r, 0)),
        out_shape=jax.ShapeDtypeStruct((n_rows, d), jnp.float32),
    )(agg, u_prev, cnt, b_pad)


def kernel(x, edge_index, W0, b0, g0, be0, W1, b1, g1, be1, W2, b2):
    n, _ = x.shape
    e = edge_index.shape[1]
    d_cls = W2.shape[1]

    src = edge_index[0].astype(jnp.int32)
    dst = edge_index[1].astype(jnp.int32)
    packed = src + (dst << PACK_SHIFT)
    fillv = n + (n << PACK_SHIFT)

    nchunk = -(-e // (NS * CHUNK))
    nchunk += nchunk % 2
    epw = nchunk * CHUNK
    pad_s = jnp.full((NS * epw - e,), fillv, jnp.int32)
    packed_s = jnp.concatenate([packed, pad_s]).reshape(NS, epw)

    nchunk_c = -(-e // (NW * CHUNK))
    epw_c = nchunk_c * CHUNK
    pad_c = jnp.full((NW * epw_c - e,), fillv, jnp.int32)
    packed_w = jnp.concatenate([packed, pad_c]).reshape(NW, epw_c)

    x_pad = jnp.pad(x, ((0, NODE_PAD - n), (0, 0)))
    zeros128 = jnp.zeros((RPT, 128), jnp.float32)
    zeros64h = jnp.zeros((RPT, 64), jnp.float32)
    zeros32h = jnp.zeros((RPT, 32), jnp.float32)
    iota_rows = jnp.arange(HROWS, dtype=jnp.int32).reshape(HROWS // 128, 128)
    W2p = jnp.pad(W2, ((0, 0), (0, 64 - d_cls)))
    b2p = jnp.concatenate([b2, jnp.full((64 - d_cls,), -1e30, jnp.float32)])

    t0 = _tc_mm(x_pad, W0)
    cnt = _sc_count(packed_w, iota_rows, zeros128)
    cnt = cnt.reshape(NC, NODE_PAD, CNT_W)
    u0 = _tc_scale(t0, cnt)
    a0 = _sc_agg(u0, packed_s, zeros64h)
    u1 = _tc_layer(a0, u0, cnt, b0, g0, be0, W1)
    a1 = _sc_agg(u1, packed_s, zeros64h)
    u2 = _tc_layer(a1, u1, cnt, b1, g1, be1, W2p)
    a2 = _sc_agg(u2, packed_s, zeros32h)
    out = _tc_out(a2, u2, cnt, b2p, n)
    return out[:, :d_cls]


# speedup vs baseline: 1.0144x; 1.0144x over previous
import functools

import jax
import jax.numpy as jnp
from jax import lax
from jax.experimental import pallas as pl
from jax.experimental.pallas import tpu as pltpu
from jax.experimental.pallas import tpu_sc as plsc

NC = 2
NS = 16
NW = NC * NS
CHUNK = 64
NODE_PAD = 10240
RPT = NODE_PAD // NS
BN_SCALE = 1.0 / (1.0 + 1e-5) ** 0.5


CNT_W = 8


HROWS = NODE_PAD * CNT_W // 128


PACK_SHIFT = 14


def _sc_count(packed_flat, iota_rows, zeros_c):
    epw = packed_flat.shape[1]
    slc = HROWS // NS
    mrows = 128

    @functools.partial(
        pl.kernel,
        out_type=jax.ShapeDtypeStruct((NC, HROWS, 128), jnp.float32),
        mesh=plsc.VectorSubcoreMesh(core_axis_name="c", subcore_axis_name="s"),
        scratch_types=[
            pltpu.VMEM_SHARED((HROWS, 128), jnp.float32),
            pltpu.VMEM((HROWS, 128), jnp.float32),
            pltpu.VMEM((epw,), jnp.int32),
            pltpu.VMEM((HROWS // mrows, mrows), jnp.int32),
        ],
        compiler_params=pltpu.CompilerParams(needs_layout_passes=False),
    )
    def k(pk_hbm, iota_hbm, zeros_hbm, cnt_hbm, cnt_sh, hist, idx_v, rowi_v):
        c = lax.axis_index("c")
        s = lax.axis_index("s")
        w = c * NS + s
        r0 = s * slc
        pltpu.sync_copy(zeros_hbm.at[pl.ds(0, slc)], cnt_sh.at[pl.ds(r0, slc)])
        pltpu.sync_copy(zeros_hbm, hist)
        pltpu.sync_copy(pk_hbm.at[w], idx_v)
        pltpu.sync_copy(iota_hbm, rowi_v)

        ones_v = jnp.ones((16,), jnp.float32)

        def body(j, carry):
            pk = idx_v[pl.ds(j * 16, 16)]
            pos = lax.shift_right_logical(pk, PACK_SHIFT) * CNT_W
            row = lax.shift_right_logical(pos, 7)
            col = lax.bitwise_and(pos, 127)
            plsc.addupdate_scatter(hist, [row, col], ones_v)
            return carry

        lax.fori_loop(0, epw // 16, body, 0)
        plsc.subcore_barrier()

        def mbody(g, carry):
            pltpu.sync_copy(hist.at[pl.ds(g * mrows, mrows)],
                            cnt_sh.at[rowi_v.at[g]], add=True)
            return carry

        lax.fori_loop(0, HROWS // mrows, mbody, 0)
        plsc.subcore_barrier()
        pltpu.sync_copy(cnt_sh.at[pl.ds(r0, slc)],
                        cnt_hbm.at[c, pl.ds(r0, slc)])

    return k(packed_flat, iota_rows, zeros_c)


def _unpack_chunk(idx_v, j, sidx, didx):
    for l in range(CHUNK // 16):
        pk = idx_v[pl.ds(j * CHUNK + l * 16, 16)]
        sidx[pl.ds(l * 16, 16)] = lax.bitwise_and(pk, (1 << PACK_SHIFT) - 1)
        didx[pl.ds(l * 16, 16)] = lax.shift_right_logical(pk, PACK_SHIFT)


def _sc_agg(u, packed_s, zeros_c):
    epw = packed_s.shape[1]
    nchunk = epw // CHUNK
    d = u.shape[1]
    dh = d // 2
    assert nchunk % 2 == 0
    npair = nchunk // 2

    @functools.partial(
        pl.kernel,
        out_type=jax.ShapeDtypeStruct((NODE_PAD, d), jnp.float32),
        mesh=plsc.VectorSubcoreMesh(core_axis_name="c", subcore_axis_name="s"),
        compiler_params=pltpu.CompilerParams(use_tc_tiling_on_sc=False),
        scratch_types=[
            pltpu.VMEM_SHARED((NODE_PAD, dh), jnp.float32),
            pltpu.VMEM_SHARED((NODE_PAD, dh), jnp.float32),
            pltpu.VMEM((epw,), jnp.int32),
            pltpu.VMEM((CHUNK,), jnp.int32),
            pltpu.VMEM((CHUNK,), jnp.int32),
            pltpu.VMEM((CHUNK,), jnp.int32),
            pltpu.VMEM((CHUNK,), jnp.int32),
            pltpu.VMEM((CHUNK, dh), jnp.float32),
            pltpu.VMEM((CHUNK, dh), jnp.float32),
            pltpu.SemaphoreType.DMA,
            pltpu.SemaphoreType.DMA,
        ],
    )
    def k(u_hbm, pk_hbm, z_hbm, agg_hbm, u_sh, agg_sh,
          idx_v, sidx0, didx0, sidx1, didx1, rows0, rows1, sem0, sem1):
        c = lax.axis_index("c")
        s = lax.axis_index("s")
        r0 = s * RPT
        c0 = c * dh
        pltpu.sync_copy(u_hbm.at[pl.ds(r0, RPT), pl.ds(c0, dh)],
                        u_sh.at[pl.ds(r0, RPT)])
        pltpu.sync_copy(z_hbm, agg_sh.at[pl.ds(r0, RPT)])
        pltpu.sync_copy(pk_hbm.at[s], idx_v)
        plsc.subcore_barrier()

        _unpack_chunk(idx_v, 0, sidx0, didx0)
        pltpu.async_copy(u_sh.at[sidx0], rows0, sem0)

        def body(i, carry):
            j = 2 * i
            _unpack_chunk(idx_v, j + 1, sidx1, didx1)
            pltpu.async_copy(u_sh.at[sidx1], rows1, sem1)
            pltpu.make_async_copy(u_sh.at[sidx0], rows0, sem0).wait()
            pltpu.sync_copy(rows0, agg_sh.at[didx0], add=True)

            @pl.when(i + 1 < npair)
            def _():
                _unpack_chunk(idx_v, j + 2, sidx0, didx0)
                pltpu.async_copy(u_sh.at[sidx0], rows0, sem0)

            pltpu.make_async_copy(u_sh.at[sidx1], rows1, sem1).wait()
            pltpu.sync_copy(rows1, agg_sh.at[didx1], add=True)
            return carry

        lax.fori_loop(0, npair, body, 0)
        plsc.subcore_barrier()
        pltpu.sync_copy(agg_sh.at[pl.ds(r0, RPT)],
                        agg_hbm.at[pl.ds(r0, RPT), pl.ds(c0, dh)])

    return k(u, packed_s, zeros_c)


BLK = 256


def _dinv_of(cnt_ref):
    return lax.rsqrt(cnt_ref[0, :, :1] + cnt_ref[1, :, :1] + 1.0)


def _tc_pre(x_pad, W0, cnt):
    d_in = x_pad.shape[1]
    d_out = W0.shape[1]

    def body(x_ref, w_ref, cnt_ref, o_ref):
        dinv = _dinv_of(cnt_ref)
        h = jnp.dot(x_ref[...], w_ref[...], preferred_element_type=jnp.float32)
        o_ref[...] = h * dinv

    return pl.pallas_call(
        body,
        grid=(NODE_PAD // BLK,),
        in_specs=[
            pl.BlockSpec((BLK, d_in), lambda r: (r, 0)),
            pl.BlockSpec((d_in, d_out), lambda r: (0, 0)),
            pl.BlockSpec((NC, BLK, CNT_W), lambda r: (0, r, 0)),
        ],
        out_specs=pl.BlockSpec((BLK, d_out), lambda r: (r, 0)),
        out_shape=jax.ShapeDtypeStruct((NODE_PAD, d_out), jnp.float32),
    )(x_pad, W0, cnt)


def _tc_layer(agg, u_prev, cnt, b, g, be, W):
    d = u_prev.shape[1]
    d_out = W.shape[1]

    def body(a_ref, u_ref, cnt_ref, b_ref, g_ref, be_ref, w_ref, o_ref):
        dinv = _dinv_of(cnt_ref)
        z = dinv * (a_ref[...] + u_ref[...]) + b_ref[...]
        z = z * (g_ref[...] * BN_SCALE) + be_ref[...]
        z = jnp.maximum(z, 0.0)
        h = jnp.dot(z, w_ref[...], preferred_element_type=jnp.float32)
        o_ref[...] = h * dinv

    return pl.pallas_call(
        body,
        grid=(NODE_PAD // BLK,),
        in_specs=[
            pl.BlockSpec((BLK, d), lambda r: (r, 0)),
            pl.BlockSpec((BLK, d), lambda r: (r, 0)),
            pl.BlockSpec((NC, BLK, CNT_W), lambda r: (0, r, 0)),
            pl.BlockSpec((d,), lambda r: (0,)),
            pl.BlockSpec((d,), lambda r: (0,)),
            pl.BlockSpec((d,), lambda r: (0,)),
            pl.BlockSpec((d, d_out), lambda r: (0, 0)),
        ],
        out_specs=pl.BlockSpec((BLK, d_out), lambda r: (r, 0)),
        out_shape=jax.ShapeDtypeStruct((NODE_PAD, d_out), jnp.float32),
    )(agg, u_prev, cnt, b, g, be, W)


def _tc_out(agg, u_prev, cnt, b_pad, n_rows):
    d = u_prev.shape[1]
    blk = 400

    def body(a_ref, u_ref, cnt_ref, b_ref, o_ref):
        dinv = _dinv_of(cnt_ref)
        z = dinv * (a_ref[...] + u_ref[...]) + b_ref[...]
        m = jnp.max(z, axis=1, keepdims=True)
        t = z - m
        lse = jnp.log(jnp.sum(jnp.exp(t), axis=1, keepdims=True))
        o_ref[...] = t - lse

    return pl.pallas_call(
        body,
        grid=(n_rows // blk,),
        in_specs=[
            pl.BlockSpec((blk, d), lambda r: (r, 0)),
            pl.BlockSpec((blk, d), lambda r: (r, 0)),
            pl.BlockSpec((NC, blk, CNT_W), lambda r: (0, r, 0)),
            pl.BlockSpec((d,), lambda r: (0,)),
        ],
        out_specs=pl.BlockSpec((blk, d), lambda r: (r, 0)),
        out_shape=jax.ShapeDtypeStruct((n_rows, d), jnp.float32),
    )(agg, u_prev, cnt, b_pad)


def kernel(x, edge_index, W0, b0, g0, be0, W1, b1, g1, be1, W2, b2):
    n, _ = x.shape
    e = edge_index.shape[1]
    d_cls = W2.shape[1]

    src = edge_index[0].astype(jnp.int32)
    dst = edge_index[1].astype(jnp.int32)
    packed = src + (dst << PACK_SHIFT)
    fillv = n + (n << PACK_SHIFT)

    nchunk = -(-e // (NS * CHUNK))
    nchunk += nchunk % 2
    epw = nchunk * CHUNK
    pad_s = jnp.full((NS * epw - e,), fillv, jnp.int32)
    packed_s = jnp.concatenate([packed, pad_s]).reshape(NS, epw)

    nchunk_c = -(-e // (NW * CHUNK))
    epw_c = nchunk_c * CHUNK
    pad_c = jnp.full((NW * epw_c - e,), fillv, jnp.int32)
    packed_w = jnp.concatenate([packed, pad_c]).reshape(NW, epw_c)

    x_pad = jnp.pad(x, ((0, NODE_PAD - n), (0, 0)))
    zeros128 = jnp.zeros((RPT, 128), jnp.float32)
    zeros64h = jnp.zeros((RPT, 64), jnp.float32)
    zeros32h = jnp.zeros((RPT, 32), jnp.float32)
    iota_rows = jnp.arange(HROWS, dtype=jnp.int32).reshape(HROWS // 128, 128)
    W2p = jnp.pad(W2, ((0, 0), (0, 64 - d_cls)))
    b2p = jnp.concatenate([b2, jnp.full((64 - d_cls,), -1e30, jnp.float32)])

    cnt = _sc_count(packed_w, iota_rows, zeros128)
    cnt = cnt.reshape(NC, NODE_PAD, CNT_W)
    u0 = _tc_pre(x_pad, W0, cnt)
    a0 = _sc_agg(u0, packed_s, zeros64h)
    u1 = _tc_layer(a0, u0, cnt, b0, g0, be0, W1)
    a1 = _sc_agg(u1, packed_s, zeros64h)
    u2 = _tc_layer(a1, u1, cnt, b1, g1, be1, W2p)
    a2 = _sc_agg(u2, packed_s, zeros32h)
    out = _tc_out(a2, u2, cnt, b2p, n)
    return out[:, :d_cls]


# speedup vs baseline: 1.0761x; 1.0609x over previous
import functools

import jax
import jax.numpy as jnp
from jax import lax
from jax.experimental import pallas as pl
from jax.experimental.pallas import tpu as pltpu
from jax.experimental.pallas import tpu_sc as plsc

NC = 2
NS = 16
NW = NC * NS
CHUNK = 64
NODE_PAD = 10240
RPT = NODE_PAD // NS
BN_SCALE = 1.0 / (1.0 + 1e-5) ** 0.5


CNT_W = 8


HROWS = NODE_PAD * CNT_W // 128


PACK_SHIFT = 14


def _sc_count(packed_flat, iota_rows, zeros_c):
    epw = packed_flat.shape[1]
    slc = HROWS // NS
    mrows = 128

    @functools.partial(
        pl.kernel,
        out_type=jax.ShapeDtypeStruct((NC, HROWS, 128), jnp.float32),
        mesh=plsc.VectorSubcoreMesh(core_axis_name="c", subcore_axis_name="s"),
        scratch_types=[
            pltpu.VMEM_SHARED((HROWS, 128), jnp.float32),
            pltpu.VMEM((HROWS, 128), jnp.float32),
            pltpu.VMEM((epw,), jnp.int32),
            pltpu.VMEM((HROWS // mrows, mrows), jnp.int32),
        ],
        compiler_params=pltpu.CompilerParams(needs_layout_passes=False),
    )
    def k(pk_hbm, iota_hbm, zeros_hbm, cnt_hbm, cnt_sh, hist, idx_v, rowi_v):
        c = lax.axis_index("c")
        s = lax.axis_index("s")
        w = c * NS + s
        r0 = s * slc
        pltpu.sync_copy(zeros_hbm.at[pl.ds(0, slc)], cnt_sh.at[pl.ds(r0, slc)])
        pltpu.sync_copy(zeros_hbm, hist)
        pltpu.sync_copy(pk_hbm.at[w], idx_v)
        pltpu.sync_copy(iota_hbm, rowi_v)

        ones_v = jnp.ones((16,), jnp.float32)

        def body(j, carry):
            pk = idx_v[pl.ds(j * 16, 16)]
            pos = lax.shift_right_logical(pk, PACK_SHIFT) * CNT_W
            row = lax.shift_right_logical(pos, 7)
            col = lax.bitwise_and(pos, 127)
            plsc.addupdate_scatter(hist, [row, col], ones_v)
            return carry

        lax.fori_loop(0, epw // 16, body, 0)
        plsc.subcore_barrier()

        def mbody(g, carry):
            pltpu.sync_copy(hist.at[pl.ds(g * mrows, mrows)],
                            cnt_sh.at[rowi_v.at[g]], add=True)
            return carry

        lax.fori_loop(0, HROWS // mrows, mbody, 0)
        plsc.subcore_barrier()
        pltpu.sync_copy(cnt_sh.at[pl.ds(r0, slc)],
                        cnt_hbm.at[c, pl.ds(r0, slc)])

    return k(packed_flat, iota_rows, zeros_c)


def _unpack_chunk(idx_v, j, sidx, didx):
    for l in range(CHUNK // 16):
        pk = idx_v[pl.ds(j * CHUNK + l * 16, 16)]
        sidx[pl.ds(l * 16, 16)] = lax.bitwise_and(pk, (1 << PACK_SHIFT) - 1)
        didx[pl.ds(l * 16, 16)] = lax.shift_right_logical(pk, PACK_SHIFT)


def _sc_agg(u, packed_s, zeros_c):
    epw = packed_s.shape[1]
    nchunk = epw // CHUNK
    d = u.shape[1]
    dh = d // 2
    assert nchunk % 2 == 0
    npair = nchunk // 2

    @functools.partial(
        pl.kernel,
        out_type=jax.ShapeDtypeStruct((NODE_PAD, d), jnp.float32),
        mesh=plsc.VectorSubcoreMesh(core_axis_name="c", subcore_axis_name="s"),
        compiler_params=pltpu.CompilerParams(use_tc_tiling_on_sc=False),
        scratch_types=[
            pltpu.VMEM_SHARED((NODE_PAD, dh), jnp.float32),
            pltpu.VMEM_SHARED((NODE_PAD, dh), jnp.float32),
            pltpu.VMEM((epw,), jnp.int32),
            pltpu.VMEM((CHUNK,), jnp.int32),
            pltpu.VMEM((CHUNK,), jnp.int32),
            pltpu.VMEM((CHUNK,), jnp.int32),
            pltpu.VMEM((CHUNK,), jnp.int32),
            pltpu.VMEM((CHUNK, dh), jnp.float32),
            pltpu.VMEM((CHUNK, dh), jnp.float32),
            pltpu.SemaphoreType.DMA,
            pltpu.SemaphoreType.DMA,
        ],
    )
    def k(u_hbm, pk_hbm, z_hbm, agg_hbm, u_sh, agg_sh,
          idx_v, sidx0, didx0, sidx1, didx1, rows0, rows1, sem0, sem1):
        c = lax.axis_index("c")
        s = lax.axis_index("s")
        r0 = s * RPT
        c0 = c * dh
        pltpu.sync_copy(u_hbm.at[pl.ds(r0, RPT), pl.ds(c0, dh)],
                        u_sh.at[pl.ds(r0, RPT)])
        pltpu.sync_copy(z_hbm, agg_sh.at[pl.ds(r0, RPT)])
        pltpu.sync_copy(pk_hbm.at[s], idx_v)
        plsc.subcore_barrier()

        _unpack_chunk(idx_v, 0, sidx0, didx0)
        pltpu.async_copy(u_sh.at[sidx0], rows0, sem0)

        def body(i, carry):
            j = 2 * i
            _unpack_chunk(idx_v, j + 1, sidx1, didx1)
            pltpu.async_copy(u_sh.at[sidx1], rows1, sem1)
            pltpu.make_async_copy(u_sh.at[sidx0], rows0, sem0).wait()
            pltpu.sync_copy(rows0, agg_sh.at[didx0], add=True)

            @pl.when(i + 1 < npair)
            def _():
                _unpack_chunk(idx_v, j + 2, sidx0, didx0)
                pltpu.async_copy(u_sh.at[sidx0], rows0, sem0)

            pltpu.make_async_copy(u_sh.at[sidx1], rows1, sem1).wait()
            pltpu.sync_copy(rows1, agg_sh.at[didx1], add=True)
            return carry

        lax.fori_loop(0, npair, body, 0)
        plsc.subcore_barrier()
        pltpu.sync_copy(agg_sh.at[pl.ds(r0, RPT)],
                        agg_hbm.at[pl.ds(r0, RPT), pl.ds(c0, dh)])

    return k(u, packed_s, zeros_c)


BLK = 512


def _dinv_of(cnt_ref):
    return lax.rsqrt(cnt_ref[0, :, :1] + cnt_ref[1, :, :1] + 1.0)


def _tc_pre(x_pad, W0, cnt):
    d_in = x_pad.shape[1]
    d_out = W0.shape[1]

    def body(x_ref, w_ref, cnt_ref, o_ref):
        dinv = _dinv_of(cnt_ref)
        h = jnp.dot(x_ref[...], w_ref[...], preferred_element_type=jnp.float32)
        o_ref[...] = h * dinv

    return pl.pallas_call(
        body,
        grid=(NODE_PAD // BLK,),
        in_specs=[
            pl.BlockSpec((BLK, d_in), lambda r: (r, 0)),
            pl.BlockSpec((d_in, d_out), lambda r: (0, 0)),
            pl.BlockSpec((NC, BLK, CNT_W), lambda r: (0, r, 0)),
        ],
        out_specs=pl.BlockSpec((BLK, d_out), lambda r: (r, 0)),
        out_shape=jax.ShapeDtypeStruct((NODE_PAD, d_out), jnp.float32),
    )(x_pad, W0, cnt)


def _tc_layer(agg, u_prev, cnt, b, g, be, W):
    d = u_prev.shape[1]
    d_out = W.shape[1]

    def body(a_ref, u_ref, cnt_ref, b_ref, g_ref, be_ref, w_ref, o_ref):
        dinv = _dinv_of(cnt_ref)
        z = dinv * (a_ref[...] + u_ref[...]) + b_ref[...]
        z = z * (g_ref[...] * BN_SCALE) + be_ref[...]
        z = jnp.maximum(z, 0.0)
        h = jnp.dot(z, w_ref[...], preferred_element_type=jnp.float32)
        o_ref[...] = h * dinv

    return pl.pallas_call(
        body,
        grid=(NODE_PAD // BLK,),
        in_specs=[
            pl.BlockSpec((BLK, d), lambda r: (r, 0)),
            pl.BlockSpec((BLK, d), lambda r: (r, 0)),
            pl.BlockSpec((NC, BLK, CNT_W), lambda r: (0, r, 0)),
            pl.BlockSpec((d,), lambda r: (0,)),
            pl.BlockSpec((d,), lambda r: (0,)),
            pl.BlockSpec((d,), lambda r: (0,)),
            pl.BlockSpec((d, d_out), lambda r: (0, 0)),
        ],
        out_specs=pl.BlockSpec((BLK, d_out), lambda r: (r, 0)),
        out_shape=jax.ShapeDtypeStruct((NODE_PAD, d_out), jnp.float32),
    )(agg, u_prev, cnt, b, g, be, W)


def _tc_out(agg, u_prev, cnt, b_pad, n_rows):
    d = u_prev.shape[1]
    blk = 400

    def body(a_ref, u_ref, cnt_ref, b_ref, o_ref):
        dinv = _dinv_of(cnt_ref)
        z = dinv * (a_ref[...] + u_ref[...]) + b_ref[...]
        m = jnp.max(z, axis=1, keepdims=True)
        t = z - m
        lse = jnp.log(jnp.sum(jnp.exp(t), axis=1, keepdims=True))
        o_ref[...] = t - lse

    return pl.pallas_call(
        body,
        grid=(n_rows // blk,),
        in_specs=[
            pl.BlockSpec((blk, d), lambda r: (r, 0)),
            pl.BlockSpec((blk, d), lambda r: (r, 0)),
            pl.BlockSpec((NC, blk, CNT_W), lambda r: (0, r, 0)),
            pl.BlockSpec((d,), lambda r: (0,)),
        ],
        out_specs=pl.BlockSpec((blk, d), lambda r: (r, 0)),
        out_shape=jax.ShapeDtypeStruct((n_rows, d), jnp.float32),
    )(agg, u_prev, cnt, b_pad)


def kernel(x, edge_index, W0, b0, g0, be0, W1, b1, g1, be1, W2, b2):
    n, _ = x.shape
    e = edge_index.shape[1]
    d_cls = W2.shape[1]

    src = edge_index[0].astype(jnp.int32)
    dst = edge_index[1].astype(jnp.int32)
    packed = src + (dst << PACK_SHIFT)
    fillv = n + (n << PACK_SHIFT)

    nchunk = -(-e // (NS * CHUNK))
    nchunk += nchunk % 2
    epw = nchunk * CHUNK
    pad_s = jnp.full((NS * epw - e,), fillv, jnp.int32)
    packed_s = jnp.concatenate([packed, pad_s]).reshape(NS, epw)

    nchunk_c = -(-e // (NW * CHUNK))
    epw_c = nchunk_c * CHUNK
    pad_c = jnp.full((NW * epw_c - e,), fillv, jnp.int32)
    packed_w = jnp.concatenate([packed, pad_c]).reshape(NW, epw_c)

    x_pad = jnp.pad(x, ((0, NODE_PAD - n), (0, 0)))
    zeros128 = jnp.zeros((RPT, 128), jnp.float32)
    zeros64h = jnp.zeros((RPT, 64), jnp.float32)
    zeros32h = jnp.zeros((RPT, 32), jnp.float32)
    iota_rows = jnp.arange(HROWS, dtype=jnp.int32).reshape(HROWS // 128, 128)
    W2p = jnp.pad(W2, ((0, 0), (0, 64 - d_cls)))
    b2p = jnp.concatenate([b2, jnp.full((64 - d_cls,), -1e30, jnp.float32)])

    cnt = _sc_count(packed_w, iota_rows, zeros128)
    cnt = cnt.reshape(NC, NODE_PAD, CNT_W)
    u0 = _tc_pre(x_pad, W0, cnt)
    a0 = _sc_agg(u0, packed_s, zeros64h)
    u1 = _tc_layer(a0, u0, cnt, b0, g0, be0, W1)
    a1 = _sc_agg(u1, packed_s, zeros64h)
    u2 = _tc_layer(a1, u1, cnt, b1, g1, be1, W2p)
    a2 = _sc_agg(u2, packed_s, zeros32h)
    out = _tc_out(a2, u2, cnt, b2p, n)
    return out[:, :d_cls]
